# baseline (speedup 1.0000x reference)
import functools

import jax
import jax.numpy as jnp
from jax import lax
from jax.experimental import pallas as pl
from jax.experimental.pallas import tpu as pltpu
from jax.experimental.pallas import tpu_sc as plsc

_NC = 2
_NS = 16
_NW = _NC * _NS


def _sc_body(u_in, i_in, u_out, i_out, sem_u, sem_i, sem_t):
    wid = lax.axis_index("s") * _NC + lax.axis_index("c")
    un = (u_in.shape[0] // _NW) & ~7
    im = (i_in.shape[0] // _NW) & ~7
    ut, it = u_in.shape[0] - un * _NW, i_in.shape[0] - im * _NW
    ub = wid * un
    ib = wid * im
    cu = pltpu.async_copy(u_in.at[pl.ds(ub, un)], u_out.at[pl.ds(ub, un)], sem_u)
    ci = pltpu.async_copy(i_in.at[pl.ds(ib, im)], i_out.at[pl.ds(ib, im)], sem_i)

    @pl.when(wid == 0)
    def _tails():
        s = pl.ds(un * _NW, ut)
        t = pl.ds(im * _NW, it)
        c1 = pltpu.async_copy(u_in.at[s], u_out.at[s], sem_t)
        c2 = pltpu.async_copy(i_in.at[t], i_out.at[t], sem_t)
        c1.wait()
        c2.wait()

    cu.wait()
    ci.wait()


def kernel(perturb, user_emb, item_emb):
    del perturb
    mesh = plsc.VectorSubcoreMesh(core_axis_name="c", subcore_axis_name="s")
    run = functools.partial(
        pl.kernel,
        mesh=mesh,
        out_type=[
            jax.ShapeDtypeStruct(user_emb.shape, user_emb.dtype),
            jax.ShapeDtypeStruct(item_emb.shape, item_emb.dtype),
        ],
        scratch_types=[pltpu.SemaphoreType.DMA, pltpu.SemaphoreType.DMA,
                       pltpu.SemaphoreType.DMA],
    )(_sc_body)
    u, i = run(user_emb, item_emb)
    return (u, i)


# speedup vs baseline: 15.3638x; 15.3638x over previous
import functools

import jax
import jax.numpy as jnp
from jax import lax
from jax.experimental import pallas as pl
from jax.experimental.pallas import tpu as pltpu
from jax.experimental.pallas import tpu_sc as plsc

_NC = 2
_NS = 16
_NW = _NC * _NS
_D = 64
_UCH = 504
_ICH = 312


def _stream_range(src, dst, base, chunk, nchunks, bufs, isems, osems):
    def start_in(c, b):
        r = base + c * chunk
        pltpu.make_async_copy(
            src.at[pl.ds(r, chunk)], bufs[b].at[pl.ds(0, chunk)],
            isems.at[b]).start()

    def wait_in(c, b):
        r = base + c * chunk
        pltpu.make_async_copy(
            src.at[pl.ds(r, chunk)], bufs[b].at[pl.ds(0, chunk)],
            isems.at[b]).wait()

    def start_out(c, b):
        r = base + c * chunk
        pltpu.make_async_copy(
            bufs[b].at[pl.ds(0, chunk)], dst.at[pl.ds(r, chunk)],
            osems.at[b]).start()

    def wait_out(c, b):
        r = base + c * chunk
        pltpu.make_async_copy(
            bufs[b].at[pl.ds(0, chunk)], dst.at[pl.ds(r, chunk)],
            osems.at[b]).wait()

    start_in(0, 0)
    for c in range(nchunks):
        b = c & 1
        nb = 1 - b
        if c + 1 < nchunks:
            if c >= 1:
                wait_out(c - 1, nb)
            start_in(c + 1, nb)
        wait_in(c, b)
        start_out(c, b)
    if nchunks >= 2:
        wait_out(nchunks - 2, (nchunks - 2) & 1)
    wait_out(nchunks - 1, (nchunks - 1) & 1)


def _sc_body(u_in, i_in, u_out, i_out, buf0, buf1, isems, osems, sem_t):
    wid = lax.axis_index("s") * _NC + lax.axis_index("c")
    un = _UCH * (((u_in.shape[0] // _NW) & ~7) // _UCH)
    im = _ICH * (((i_in.shape[0] // _NW) & ~7) // _ICH)
    bufs = (buf0, buf1)
    _stream_range(u_in, u_out, wid * un, _UCH, un // _UCH, bufs, isems, osems)
    _stream_range(i_in, i_out, wid * im, _ICH, im // _ICH, bufs, isems, osems)

    @pl.when(wid == 0)
    def _tails():
        for src, dst, lo, n in (
            (u_in, u_out, un * _NW, u_in.shape[0] - un * _NW),
            (i_in, i_out, im * _NW, i_in.shape[0] - im * _NW),
        ):
            if n == 0:
                continue
            c1 = pltpu.make_async_copy(
                src.at[pl.ds(lo, n)], buf0.at[pl.ds(0, n)], sem_t)
            c1.start()
            c1.wait()
            c2 = pltpu.make_async_copy(
                buf0.at[pl.ds(0, n)], dst.at[pl.ds(lo, n)], sem_t)
            c2.start()
            c2.wait()


def kernel(perturb, user_emb, item_emb):
    del perturb
    mesh = plsc.VectorSubcoreMesh(core_axis_name="c", subcore_axis_name="s")
    run = functools.partial(
        pl.kernel,
        mesh=mesh,
        out_type=[
            jax.ShapeDtypeStruct(user_emb.shape, user_emb.dtype),
            jax.ShapeDtypeStruct(item_emb.shape, item_emb.dtype),
        ],
        scratch_types=[
            pltpu.VMEM((_UCH, _D), jnp.float32),
            pltpu.VMEM((_UCH, _D), jnp.float32),
            pltpu.SemaphoreType.DMA((2,)),
            pltpu.SemaphoreType.DMA((2,)),
            pltpu.SemaphoreType.DMA,
        ],
    )(_sc_body)
    u, i = run(user_emb, item_emb)
    return (u, i)


# speedup vs baseline: 99.3235x; 6.4648x over previous
import jax
import jax.numpy as jnp
from jax.experimental import pallas as pl
from jax.experimental.pallas import tpu as pltpu


def _copy_body(i_ref, o_ref):
    o_ref[...] = i_ref[...]


def _copy2d(x, blk):
    n = x.shape[1]
    grid = (n + blk - 1) // blk
    return pl.pallas_call(
        _copy_body,
        grid=(grid,),
        in_specs=[pl.BlockSpec((x.shape[0], blk), lambda g: (0, g))],
        out_specs=pl.BlockSpec((x.shape[0], blk), lambda g: (0, g)),
        out_shape=jax.ShapeDtypeStruct(x.shape, x.dtype),
    )(x)


def kernel(perturb, user_emb, item_emb):
    del perturb
    u = _copy2d(user_emb.T, 16000).T
    i = _copy2d(item_emb.T, 12800).T
    return (u, i)
